# baseline (speedup 1.0000x reference)
import jax
import jax.numpy as jnp
from jax import lax
from jax.experimental import pallas as pl
from jax.experimental.pallas import tpu as pltpu

_K = 1024
_TN = 1024


def _split3(x):
    hi = x.astype(jnp.bfloat16)
    r1 = x - hi.astype(jnp.float32)
    mid = r1.astype(jnp.bfloat16)
    lo = (r1 - mid.astype(jnp.float32)).astype(jnp.bfloat16)
    return hi, mid, lo


def _iter_body(feat_ref, hi_ref, mid_ref, lo_ref, cent_ref,
               labels_ref, newc_ref, sums_acc, counts_acc, coln_acc):
    i = pl.program_id(0)
    nt = pl.num_programs(0)
    ft = feat_ref[...]
    c = cent_ref[...]
    tn, d = ft.shape
    k = c.shape[0]

    @pl.when(i == 0)
    def _init():
        sums_acc[...] = jnp.zeros_like(sums_acc)
        counts_acc[...] = jnp.zeros_like(counts_acc)
        coln_acc[0:1, :] = lax.dot_general(
            jnp.ones((1, d), jnp.float32), c * c, (((1,), (1,)), ((), ())),
            precision=lax.Precision.HIGHEST)

    rown = jnp.sum(ft * ft, axis=1, keepdims=True)
    coln = coln_acc[0:1, :]
    fc = lax.dot_general(ft, c, (((1,), (1,)), ((), ())),
                         precision=lax.Precision.DEFAULT)
    sq = (rown - 2.0 * fc) + coln
    iota = lax.broadcasted_iota(jnp.int32, (tn, k), 1)
    m = jnp.min(sq, axis=1, keepdims=True)
    labels = jnp.min(jnp.where(sq == m, iota, k), axis=1)
    labels_ref[...] = labels.reshape(1, 1, tn)

    onehot = (labels[:, None] == iota).astype(jnp.bfloat16)
    dn = (((0,), (0,)), ((), ()))
    sums_acc[...] += (
        lax.dot_general(onehot, hi_ref[...], dn,
                        preferred_element_type=jnp.float32)
        + lax.dot_general(onehot, mid_ref[...], dn,
                          preferred_element_type=jnp.float32)
        + lax.dot_general(onehot, lo_ref[...], dn,
                          preferred_element_type=jnp.float32))
    counts_acc[...] += lax.dot_general(
        onehot, jnp.ones((tn, 128), jnp.bfloat16), dn,
        preferred_element_type=jnp.float32)

    @pl.when(i == nt - 1)
    def _update():
        counts = counts_acc[:, 0:1]
        sums = sums_acc[...]
        newc_ref[...] = jnp.where(counts > 0.0,
                                  sums / jnp.maximum(counts, 1.0), 0.0)


def _kmeans_iter(features, fhi, fmid, flo, centroids):
    n, d = features.shape
    k = centroids.shape[0]
    nt = n // _TN
    labels3, newc = pl.pallas_call(
        _iter_body,
        grid=(nt,),
        in_specs=[
            pl.BlockSpec((_TN, d), lambda i: (i, 0)),
            pl.BlockSpec((_TN, d), lambda i: (i, 0)),
            pl.BlockSpec((_TN, d), lambda i: (i, 0)),
            pl.BlockSpec((_TN, d), lambda i: (i, 0)),
            pl.BlockSpec((k, d), lambda i: (0, 0)),
        ],
        out_specs=[
            pl.BlockSpec((1, 1, _TN), lambda i: (i, 0, 0)),
            pl.BlockSpec((k, d), lambda i: (0, 0)),
        ],
        out_shape=[
            jax.ShapeDtypeStruct((nt, 1, _TN), jnp.int32),
            jax.ShapeDtypeStruct((k, d), jnp.float32),
        ],
        scratch_shapes=[
            pltpu.VMEM((k, d), jnp.float32),
            pltpu.VMEM((k, 128), jnp.float32),
            pltpu.VMEM((8, k), jnp.float32),
        ],
    )(features, fhi, fmid, flo, centroids)
    return labels3.reshape(n), newc


def _final_body(feat_ref, cent_ref, labels_ref, ff_ref, dsum_ref, acc_ref):
    i = pl.program_id(0)
    nt = pl.num_programs(0)
    ft = feat_ref[...]
    c = cent_ref[...]
    tn, d = ft.shape
    k = c.shape[0]
    labels = labels_ref[0, 0, :]

    onehot = (labels[:, None] ==
              lax.broadcasted_iota(jnp.int32, (tn, k), 1)).astype(jnp.bfloat16)
    chi, cmid, clo = _split3(c)
    dn = (((1,), (0,)), ((), ()))
    ff = (lax.dot_general(onehot, chi, dn, preferred_element_type=jnp.float32)
          + lax.dot_general(onehot, cmid, dn, preferred_element_type=jnp.float32)
          + lax.dot_general(onehot, clo, dn, preferred_element_type=jnp.float32))
    ff_ref[...] = ff

    diff = ft - ff
    part = jnp.sum(diff * diff)

    @pl.when(i == 0)
    def _init():
        acc_ref[0, 0] = 0.0

    acc_ref[0, 0] += part

    @pl.when(i == nt - 1)
    def _write():
        dsum_ref[0, 0] = acc_ref[0, 0]


def _finalize(features, centroids, labels):
    n, d = features.shape
    k = centroids.shape[0]
    nt = n // _TN
    labels3 = labels.reshape(nt, 1, _TN)
    ff, dsum = pl.pallas_call(
        _final_body,
        grid=(nt,),
        in_specs=[
            pl.BlockSpec((_TN, d), lambda i: (i, 0)),
            pl.BlockSpec((k, d), lambda i: (0, 0)),
            pl.BlockSpec((1, 1, _TN), lambda i: (i, 0, 0)),
        ],
        out_specs=[
            pl.BlockSpec((_TN, d), lambda i: (i, 0)),
            pl.BlockSpec(memory_space=pltpu.SMEM),
        ],
        out_shape=[
            jax.ShapeDtypeStruct((n, d), jnp.float32),
            jax.ShapeDtypeStruct((1, 1), jnp.float32),
        ],
        scratch_shapes=[pltpu.SMEM((1, 1), jnp.float32)],
    )(features, centroids, labels3)
    return ff, dsum[0, 0]


def kernel(features, max_iters):
    n, d = features.shape
    perm = jax.random.permutation(jax.random.key(1), n)[:_K]
    cent0 = features[perm]
    labels0 = jnp.zeros((n,), jnp.int32)
    fhi = lax.optimization_barrier(features.astype(jnp.bfloat16))
    r1 = features - fhi.astype(jnp.float32)
    fmid = lax.optimization_barrier(r1.astype(jnp.bfloat16))
    flo = lax.optimization_barrier(
        (r1 - fmid.astype(jnp.float32)).astype(jnp.bfloat16))

    def body(_, carry):
        cent, _labels = carry
        labels, newc = _kmeans_iter(features, fhi, fmid, flo, cent)
        return newc, labels

    cent, labels = lax.fori_loop(0, max_iters, body, (cent0, labels0))
    ff, dsum = _finalize(features, cent, labels)
    differences = dsum / jnp.float32(n * d)
    return ff, labels, differences


# speedup vs baseline: 1.1044x; 1.1044x over previous
import jax
import jax.numpy as jnp
from jax import lax
from jax.experimental import pallas as pl
from jax.experimental.pallas import tpu as pltpu

_K = 1024
_TN = 1024


def _split3(x):
    hi = x.astype(jnp.bfloat16)
    r1 = x - hi.astype(jnp.float32)
    mid = r1.astype(jnp.bfloat16)
    lo = (r1 - mid.astype(jnp.float32)).astype(jnp.bfloat16)
    return hi, mid, lo


def _iter_body(feat_ref, hml_ref, cent_ref,
               labels_ref, newc_ref, sums_acc, coln_acc):
    i = pl.program_id(0)
    nt = pl.num_programs(0)
    ft = feat_ref[...]
    c = cent_ref[...]
    tn, d = ft.shape
    k = c.shape[0]

    @pl.when(i == 0)
    def _init():
        sums_acc[...] = jnp.zeros_like(sums_acc)
        coln_acc[0:1, :] = lax.dot_general(
            jnp.ones((1, d), jnp.float32), c * c, (((1,), (1,)), ((), ())),
            precision=lax.Precision.HIGHEST)

    rown = jnp.sum(ft * ft, axis=1, keepdims=True)
    coln = coln_acc[0:1, :]
    fc = lax.dot_general(ft, c, (((1,), (1,)), ((), ())),
                         precision=lax.Precision.DEFAULT)
    sq = (rown - 2.0 * fc) + coln
    labels = jnp.argmin(sq, axis=1).astype(jnp.int32)
    labels_ref[...] = labels.reshape(1, 1, tn)

    onehot = (labels[:, None] ==
              lax.broadcasted_iota(jnp.int32, (tn, k), 1)).astype(jnp.bfloat16)
    dn = (((0,), (0,)), ((), ()))
    sums_acc[...] += lax.dot_general(onehot, hml_ref[...], dn,
                                     preferred_element_type=jnp.float32)

    @pl.when(i == nt - 1)
    def _update():
        sa = sums_acc[...]
        sums = (sa[:, 0:d] + sa[:, d:2 * d]) + sa[:, 2 * d:3 * d]
        counts = sa[:, 3 * d:3 * d + 1]
        newc_ref[...] = jnp.where(counts > 0.0,
                                  sums / jnp.maximum(counts, 1.0), 0.0)


def _kmeans_iter(features, hml, centroids):
    n, d = features.shape
    k = centroids.shape[0]
    w = hml.shape[1]
    nt = n // _TN
    labels3, newc = pl.pallas_call(
        _iter_body,
        grid=(nt,),
        in_specs=[
            pl.BlockSpec((_TN, d), lambda i: (i, 0)),
            pl.BlockSpec((_TN, w), lambda i: (i, 0)),
            pl.BlockSpec((k, d), lambda i: (0, 0)),
        ],
        out_specs=[
            pl.BlockSpec((1, 1, _TN), lambda i: (i, 0, 0)),
            pl.BlockSpec((k, d), lambda i: (0, 0)),
        ],
        out_shape=[
            jax.ShapeDtypeStruct((nt, 1, _TN), jnp.int32),
            jax.ShapeDtypeStruct((k, d), jnp.float32),
        ],
        scratch_shapes=[
            pltpu.VMEM((k, w), jnp.float32),
            pltpu.VMEM((8, k), jnp.float32),
        ],
    )(features, hml, centroids)
    return labels3.reshape(n), newc


def _final_body(feat_ref, cent_ref, labels_ref, ff_ref, dsum_ref, acc_ref):
    i = pl.program_id(0)
    nt = pl.num_programs(0)
    ft = feat_ref[...]
    c = cent_ref[...]
    tn, d = ft.shape
    k = c.shape[0]
    labels = labels_ref[0, 0, :]

    onehot = (labels[:, None] ==
              lax.broadcasted_iota(jnp.int32, (tn, k), 1)).astype(jnp.bfloat16)
    chi, cmid, clo = _split3(c)
    dn = (((1,), (0,)), ((), ()))
    ff = (lax.dot_general(onehot, chi, dn, preferred_element_type=jnp.float32)
          + lax.dot_general(onehot, cmid, dn, preferred_element_type=jnp.float32)
          + lax.dot_general(onehot, clo, dn, preferred_element_type=jnp.float32))
    ff_ref[...] = ff

    diff = ft - ff
    part = jnp.sum(diff * diff)

    @pl.when(i == 0)
    def _init():
        acc_ref[0, 0] = 0.0

    acc_ref[0, 0] += part

    @pl.when(i == nt - 1)
    def _write():
        dsum_ref[0, 0] = acc_ref[0, 0]


def _finalize(features, centroids, labels):
    n, d = features.shape
    k = centroids.shape[0]
    nt = n // _TN
    labels3 = labels.reshape(nt, 1, _TN)
    ff, dsum = pl.pallas_call(
        _final_body,
        grid=(nt,),
        in_specs=[
            pl.BlockSpec((_TN, d), lambda i: (i, 0)),
            pl.BlockSpec((k, d), lambda i: (0, 0)),
            pl.BlockSpec((1, 1, _TN), lambda i: (i, 0, 0)),
        ],
        out_specs=[
            pl.BlockSpec((_TN, d), lambda i: (i, 0)),
            pl.BlockSpec(memory_space=pltpu.SMEM),
        ],
        out_shape=[
            jax.ShapeDtypeStruct((n, d), jnp.float32),
            jax.ShapeDtypeStruct((1, 1), jnp.float32),
        ],
        scratch_shapes=[pltpu.SMEM((1, 1), jnp.float32)],
    )(features, centroids, labels3)
    return ff, dsum[0, 0]


def kernel(features, max_iters):
    n, d = features.shape
    perm = jax.random.permutation(jax.random.key(1), n)[:_K]
    cent0 = features[perm]
    labels0 = jnp.zeros((n,), jnp.int32)
    fhi = lax.optimization_barrier(features.astype(jnp.bfloat16))
    r1 = features - fhi.astype(jnp.float32)
    fmid = lax.optimization_barrier(r1.astype(jnp.bfloat16))
    flo = lax.optimization_barrier(
        (r1 - fmid.astype(jnp.float32)).astype(jnp.bfloat16))
    hml = jnp.concatenate(
        [fhi, fmid, flo, jnp.ones((n, 128), jnp.bfloat16)], axis=1)

    def body(_, carry):
        cent, _labels = carry
        labels, newc = _kmeans_iter(features, hml, cent)
        return newc, labels

    cent, labels = lax.fori_loop(0, max_iters, body, (cent0, labels0))
    ff, dsum = _finalize(features, cent, labels)
    differences = dsum / jnp.float32(n * d)
    return ff, labels, differences
